# Initial kernel scaffold; baseline (speedup 1.0000x reference)
import jax
import jax.numpy as jnp
from jax.experimental import pallas as pl


def kernel(inputs, emb_table):
    raise NotImplementedError("write your pallas kernel here")


# speedup vs baseline: 1.9057x; 1.9057x over previous
import jax
import jax.numpy as jnp
from jax.experimental import pallas as pl


def _add_kernel(x_ref, e_ref, o_ref):
    o_ref[...] = x_ref[...] + e_ref[...]


def kernel(inputs, emb_table):
    B, L, D = inputs.shape
    BL = 512
    grid = (L // BL, B)
    return pl.pallas_call(
        _add_kernel,
        grid=grid,
        in_specs=[
            pl.BlockSpec((1, BL, D), lambda j, b: (b, j, 0)),
            pl.BlockSpec((1, BL, D), lambda j, b: (0, j, 0)),
        ],
        out_specs=pl.BlockSpec((1, BL, D), lambda j, b: (b, j, 0)),
        out_shape=jax.ShapeDtypeStruct((B, L, D), inputs.dtype),
    )(inputs, emb_table[None])


# speedup vs baseline: 2.1160x; 1.1104x over previous
import jax
import jax.numpy as jnp
from jax.experimental import pallas as pl


def _add_kernel(x_ref, e_ref, o_ref):
    o_ref[...] = x_ref[...] + e_ref[...]


def kernel(inputs, emb_table):
    B, L, D = inputs.shape
    BL = 1024
    grid = (L // BL, B)
    return pl.pallas_call(
        _add_kernel,
        grid=grid,
        in_specs=[
            pl.BlockSpec((1, BL, D), lambda j, b: (b, j, 0)),
            pl.BlockSpec((1, BL, D), lambda j, b: (0, j, 0)),
        ],
        out_specs=pl.BlockSpec((1, BL, D), lambda j, b: (b, j, 0)),
        out_shape=jax.ShapeDtypeStruct((B, L, D), inputs.dtype),
    )(inputs, emb_table[None])


# speedup vs baseline: 2.1516x; 1.0168x over previous
import jax
import jax.numpy as jnp
from jax.experimental import pallas as pl


def _add_kernel(x_ref, e_ref, o_ref):
    o_ref[...] = x_ref[...] + e_ref[...]


def kernel(inputs, emb_table):
    B, L, D = inputs.shape
    BL = 512
    grid = (L // BL,)
    return pl.pallas_call(
        _add_kernel,
        grid=grid,
        in_specs=[
            pl.BlockSpec((B, BL, D), lambda j: (0, j, 0)),
            pl.BlockSpec((1, BL, D), lambda j: (0, j, 0)),
        ],
        out_specs=pl.BlockSpec((B, BL, D), lambda j: (0, j, 0)),
        out_shape=jax.ShapeDtypeStruct((B, L, D), inputs.dtype),
    )(inputs, emb_table[None])
